# baseline (speedup 1.0000x reference)
import functools

import jax
import jax.numpy as jnp
from jax import lax
from jax.experimental import pallas as pl
from jax.experimental.pallas import tpu as pltpu
from jax.experimental.pallas import tpu_sc as plsc

N = 10000
E = 320000
D = 128

NC = 2
NS = 16
NW = NC * NS

A_EPT = E // NW
A_K = 125
A_C = A_EPT // A_K
HC = A_C // 2

G_EPT = E // NS
G_K = 100
G_C = G_EPT // G_K

NPAD = 10240
NPT = NPAD // NS

_mesh = plsc.VectorSubcoreMesh(core_axis_name="c", subcore_axis_name="s")


@functools.partial(
    pl.kernel,
    mesh=_mesh,
    out_type=jax.ShapeDtypeStruct((NPAD,), jnp.float32),
    scratch_types=[
        pltpu.VMEM((G_C, G_K), jnp.int32),
        pltpu.VMEM((112,), jnp.float32),
        pltpu.VMEM((NPT,), jnp.float32),
        pltpu.VMEM_SHARED((NPAD,), jnp.float32),
        pltpu.SemaphoreType.DMA,
    ],
)
def _deg_kernel(src_hbm, dinv_hbm, idx_v, ones_v, dbuf_v, acc_sh, sem):
    c = lax.axis_index("c")
    s = lax.axis_index("s")

    for i in range(7):
        ones_v[pl.ds(i * 16, 16)] = jnp.full((16,), 1.0, jnp.float32)

    def _zero(i, carry):
        dbuf_v[pl.ds(i * 16, 16)] = jnp.zeros((16,), jnp.float32)
        return carry

    lax.fori_loop(0, NPT // 16, _zero, 0)
    pltpu.sync_copy(dbuf_v, acc_sh.at[pl.ds(s * NPT, NPT)])
    plsc.subcore_barrier()

    pltpu.sync_copy(src_hbm.at[s], idx_v)

    def _hd(j):
        return pltpu.make_async_copy(ones_v.at[pl.ds(0, G_K)], acc_sh.at[idx_v.at[j]], sem)

    GG = 10
    NG = G_C // GG

    def _fire(base):
        for u in range(GG):
            _hd(base + u).start(add=True)

    def _drain(base):
        for u in range(GG):
            _hd(base + u).wait()

    _fire(0)

    def _hist(g, carry):
        _fire((g + 1) * GG)
        _drain(g * GG)
        return carry

    lax.fori_loop(0, NG - 1, _hist, 0)
    _drain((NG - 1) * GG)
    plsc.subcore_barrier()

    pltpu.sync_copy(acc_sh.at[pl.ds(s * NPT, NPT)], dbuf_v)

    def _recip(i, carry):
        v = dbuf_v[pl.ds(i * 16, 16)]
        dbuf_v[pl.ds(i * 16, 16)] = 1.0 / jnp.maximum(v, 1.0)
        return carry

    lax.fori_loop(0, NPT // 16, _recip, 0)

    @pl.when(c == 0)
    def _():
        pltpu.sync_copy(dbuf_v, dinv_hbm.at[pl.ds(s * NPT, NPT)])


@functools.partial(
    pl.kernel,
    mesh=_mesh,
    out_type=jax.ShapeDtypeStruct((NC, NPAD, D), jnp.float32),
    scratch_types=[
        pltpu.VMEM((HC, A_K), jnp.int32),
        pltpu.VMEM((HC, A_K), jnp.int32),
        pltpu.VMEM((2, A_K, D), jnp.float32),
        pltpu.VMEM_SHARED((NPAD, D), jnp.float32),
        pltpu.SemaphoreType.DMA,
        pltpu.SemaphoreType.DMA,
    ],
)
def _agg_kernel(h_hbm, src_hbm, dst_hbm, out_hbm, src_v, dst_v, rows_v, acc_sh, gsem, ssem):
    c = lax.axis_index("c")
    s = lax.axis_index("s")

    def _zrow(r, carry):
        for q in range(D // 16):
            rows_v[0, r, pl.ds(q * 16, 16)] = jnp.zeros((16,), jnp.float32)
        return carry

    lax.fori_loop(0, A_K, _zrow, 0)
    for t in range(5):
        pltpu.sync_copy(rows_v.at[0], acc_sh.at[pl.ds(s * NPT + t * A_K, A_K)])
    pltpu.sync_copy(rows_v.at[0, pl.ds(0, 15)], acc_sh.at[pl.ds(s * NPT + 625, 15)])
    plsc.subcore_barrier()

    def _gd(j, b):
        return pltpu.make_async_copy(h_hbm.at[src_v.at[j]], rows_v.at[b], gsem)

    def _sd(j, b):
        return pltpu.make_async_copy(rows_v.at[b], acc_sh.at[dst_v.at[j]], ssem)

    for half in range(2):
        pltpu.sync_copy(src_hbm.at[c, s, pl.ds(half * HC, HC)], src_v)
        pltpu.sync_copy(dst_hbm.at[c, s, pl.ds(half * HC, HC)], dst_v)

        _gd(0, 0).start()
        _gd(0, 0).wait()
        _sd(0, 0).start(add=True)
        _gd(1, 1).start()

        def _body(jo, carry):
            for u in range(2):
                j = 2 * jo + 1 + u
                b = (1 + u) % 2
                _gd(j, b).wait()
                _sd(j, b).start(add=True)
                _sd(j - 1, 1 - b).wait()
                _gd(j + 1, 1 - b).start()
            return carry

        lax.fori_loop(0, (HC - 2) // 2, _body, 0)

        _gd(HC - 1, 1).wait()
        _sd(HC - 1, 1).start(add=True)
        _sd(HC - 2, 0).wait()
        _sd(HC - 1, 1).wait()

    plsc.subcore_barrier()

    pltpu.sync_copy(acc_sh.at[pl.ds(s * NPT, NPT)], out_hbm.at[c, pl.ds(s * NPT, NPT)])


R = 2000


def _lin_body(x_ref, w_ref, d_ref, o_ref):
    h = lax.dot_general(
        x_ref[...], w_ref[...], (((1,), (1,)), ((), ())),
        preferred_element_type=jnp.float32,
    )
    o_ref[...] = h * d_ref[...]


_lin1 = pl.pallas_call(
    _lin_body,
    grid=(N // R,),
    in_specs=[
        pl.BlockSpec((R, D), lambda i: (i, 0)),
        pl.BlockSpec((D, D), lambda i: (0, 0)),
        pl.BlockSpec((R, 1), lambda i: (i, 0)),
    ],
    out_specs=pl.BlockSpec((R, D), lambda i: (i, 0)),
    out_shape=jax.ShapeDtypeStruct((N, D), jnp.float32),
)


def _lin2_body(p0_ref, p1_ref, b_ref, w_ref, d_ref, o_ref):
    h = jnp.maximum(p0_ref[...] + p1_ref[...] + b_ref[...], 0.0)
    h = lax.dot_general(
        h, w_ref[...], (((1,), (1,)), ((), ())),
        preferred_element_type=jnp.float32,
    )
    o_ref[...] = h * d_ref[...]


_lin2 = pl.pallas_call(
    _lin2_body,
    grid=(N // R,),
    in_specs=[
        pl.BlockSpec((R, D), lambda i: (i, 0)),
        pl.BlockSpec((R, D), lambda i: (i, 0)),
        pl.BlockSpec((1, D), lambda i: (0, 0)),
        pl.BlockSpec((D, D), lambda i: (0, 0)),
        pl.BlockSpec((R, 1), lambda i: (i, 0)),
    ],
    out_specs=pl.BlockSpec((R, D), lambda i: (i, 0)),
    out_shape=jax.ShapeDtypeStruct((N, D), jnp.float32),
)


def _fin_body(q0_ref, q1_ref, b_ref, o_ref):
    o_ref[...] = q0_ref[...] + q1_ref[...] + b_ref[...]


_fin = pl.pallas_call(
    _fin_body,
    grid=(N // R,),
    in_specs=[
        pl.BlockSpec((R, D), lambda i: (i, 0)),
        pl.BlockSpec((R, D), lambda i: (i, 0)),
        pl.BlockSpec((1, D), lambda i: (0, 0)),
    ],
    out_specs=pl.BlockSpec((R, D), lambda i: (i, 0)),
    out_shape=jax.ShapeDtypeStruct((N, D), jnp.float32),
)


def kernel(x, edge_index, W1, b1, W2, b2):
    ei = edge_index.astype(jnp.int32)
    src, dst = ei[0], ei[1]
    src_a = src.reshape(NC, NS, A_C, A_K)
    dst_a = dst.reshape(NC, NS, A_C, A_K)
    src_g = src.reshape(NS, G_C, G_K)

    dinv = _deg_kernel(src_g)
    dcol = dinv[:N].reshape(N, 1)

    h1 = _lin1(x, W1, dcol)
    p = _agg_kernel(h1, src_a, dst_a)
    h2 = _lin2(p[0, :N], p[1, :N], b1.reshape(1, D), W2, dcol)
    q = _agg_kernel(h2, src_a, dst_a)
    return _fin(q[0, :N], q[1, :N], b2.reshape(1, D))


# speedup vs baseline: 1.1557x; 1.1557x over previous
import functools

import jax
import jax.numpy as jnp
from jax import lax
from jax.experimental import pallas as pl
from jax.experimental.pallas import tpu as pltpu
from jax.experimental.pallas import tpu_sc as plsc

N = 10000
E = 320000
D = 128

NC = 2
NS = 16
NW = NC * NS

A_EPT = E // NW
A_K = 125
A_C = A_EPT // A_K
HC = A_C // 2

G_EPT = E // NS
G_K = 100
G_C = G_EPT // G_K

NPAD = 10240
NPT = NPAD // NS

_mesh = plsc.VectorSubcoreMesh(core_axis_name="c", subcore_axis_name="s")


@functools.partial(
    pl.kernel,
    mesh=_mesh,
    out_type=jax.ShapeDtypeStruct((NPAD,), jnp.float32),
    scratch_types=[
        pltpu.VMEM((G_C, G_K), jnp.int32),
        pltpu.VMEM((112,), jnp.float32),
        pltpu.VMEM((NPT,), jnp.float32),
        pltpu.VMEM_SHARED((NPAD,), jnp.float32),
        pltpu.SemaphoreType.DMA,
    ],
)
def _deg_kernel(src_hbm, dinv_hbm, idx_v, ones_v, dbuf_v, acc_sh, sem):
    c = lax.axis_index("c")
    s = lax.axis_index("s")

    for i in range(7):
        ones_v[pl.ds(i * 16, 16)] = jnp.full((16,), 1.0, jnp.float32)

    def _zero(i, carry):
        dbuf_v[pl.ds(i * 16, 16)] = jnp.zeros((16,), jnp.float32)
        return carry

    lax.fori_loop(0, NPT // 16, _zero, 0)
    pltpu.sync_copy(dbuf_v, acc_sh.at[pl.ds(s * NPT, NPT)])
    plsc.subcore_barrier()

    pltpu.sync_copy(src_hbm.at[s], idx_v)

    def _hd(j):
        return pltpu.make_async_copy(ones_v.at[pl.ds(0, G_K)], acc_sh.at[idx_v.at[j]], sem)

    GG = 10
    NG = G_C // GG

    def _fire(base):
        for u in range(GG):
            _hd(base + u).start(add=True)

    def _drain(base):
        for u in range(GG):
            _hd(base + u).wait()

    _fire(0)

    def _hist(g, carry):
        _fire((g + 1) * GG)
        _drain(g * GG)
        return carry

    lax.fori_loop(0, NG - 1, _hist, 0)
    _drain((NG - 1) * GG)
    plsc.subcore_barrier()

    pltpu.sync_copy(acc_sh.at[pl.ds(s * NPT, NPT)], dbuf_v)

    def _recip(i, carry):
        v = dbuf_v[pl.ds(i * 16, 16)]
        dbuf_v[pl.ds(i * 16, 16)] = 1.0 / jnp.maximum(v, 1.0)
        return carry

    lax.fori_loop(0, NPT // 16, _recip, 0)

    @pl.when(c == 0)
    def _():
        pltpu.sync_copy(dbuf_v, dinv_hbm.at[pl.ds(s * NPT, NPT)])


@functools.partial(
    pl.kernel,
    mesh=_mesh,
    out_type=jax.ShapeDtypeStruct((NC, NPAD, D), jnp.float32),
    scratch_types=[
        pltpu.VMEM((HC, A_K), jnp.int32),
        pltpu.VMEM((HC, A_K), jnp.int32),
        pltpu.VMEM((2, A_K, D), jnp.float32),
        pltpu.VMEM_SHARED((NPAD, D), jnp.float32),
        pltpu.SemaphoreType.DMA,
    ],
)
def _agg_kernel(h_hbm, src_hbm, dst_hbm, out_hbm, src_v, dst_v, gbuf_v, acc_sh, gsem):
    c = lax.axis_index("c")
    s = lax.axis_index("s")

    def _zrow(r, carry):
        for q in range(D // 16):
            gbuf_v[0, r, pl.ds(q * 16, 16)] = jnp.zeros((16,), jnp.float32)
        return carry

    lax.fori_loop(0, A_K, _zrow, 0)
    for t in range(5):
        pltpu.sync_copy(gbuf_v.at[0], acc_sh.at[pl.ds(s * NPT + t * A_K, A_K)])
    pltpu.sync_copy(gbuf_v.at[0, pl.ds(0, 15)], acc_sh.at[pl.ds(s * NPT + 625, 15)])
    plsc.subcore_barrier()

    def _gd(j, b):
        return pltpu.make_async_copy(h_hbm.at[src_v.at[j]], gbuf_v.at[b], gsem)

    for half in range(2):
        pltpu.sync_copy(src_hbm.at[c, s, pl.ds(half * HC, HC)], src_v)
        pltpu.sync_copy(dst_hbm.at[c, s, pl.ds(half * HC, HC)], dst_v)

        _gd(0, 0).start()

        def _body(jo, carry):
            for b in range(2):
                j = jo * 2 + b
                if b == 0:
                    _gd(j + 1, 1).start()
                else:
                    @pl.when(jo != HC // 2 - 1)
                    def _():
                        _gd(j + 1, 0).start()
                _gd(j, b).wait()
                pltpu.sync_copy(gbuf_v.at[b], acc_sh.at[dst_v.at[j]], add=True)
            return carry

        lax.fori_loop(0, HC // 2, _body, 0)

    plsc.subcore_barrier()

    pltpu.sync_copy(acc_sh.at[pl.ds(s * NPT, NPT)], out_hbm.at[c, pl.ds(s * NPT, NPT)])


R = 2000


def _lin_body(x_ref, w_ref, d_ref, o_ref):
    h = lax.dot_general(
        x_ref[...], w_ref[...], (((1,), (1,)), ((), ())),
        preferred_element_type=jnp.float32,
    )
    o_ref[...] = h * d_ref[...]


_lin1 = pl.pallas_call(
    _lin_body,
    grid=(N // R,),
    in_specs=[
        pl.BlockSpec((R, D), lambda i: (i, 0)),
        pl.BlockSpec((D, D), lambda i: (0, 0)),
        pl.BlockSpec((R, 1), lambda i: (i, 0)),
    ],
    out_specs=pl.BlockSpec((R, D), lambda i: (i, 0)),
    out_shape=jax.ShapeDtypeStruct((N, D), jnp.float32),
)


def _lin2_body(p0_ref, p1_ref, b_ref, w_ref, d_ref, o_ref):
    h = jnp.maximum(p0_ref[...] + p1_ref[...] + b_ref[...], 0.0)
    h = lax.dot_general(
        h, w_ref[...], (((1,), (1,)), ((), ())),
        preferred_element_type=jnp.float32,
    )
    o_ref[...] = h * d_ref[...]


_lin2 = pl.pallas_call(
    _lin2_body,
    grid=(N // R,),
    in_specs=[
        pl.BlockSpec((R, D), lambda i: (i, 0)),
        pl.BlockSpec((R, D), lambda i: (i, 0)),
        pl.BlockSpec((1, D), lambda i: (0, 0)),
        pl.BlockSpec((D, D), lambda i: (0, 0)),
        pl.BlockSpec((R, 1), lambda i: (i, 0)),
    ],
    out_specs=pl.BlockSpec((R, D), lambda i: (i, 0)),
    out_shape=jax.ShapeDtypeStruct((N, D), jnp.float32),
)


def _fin_body(q0_ref, q1_ref, b_ref, o_ref):
    o_ref[...] = q0_ref[...] + q1_ref[...] + b_ref[...]


_fin = pl.pallas_call(
    _fin_body,
    grid=(N // R,),
    in_specs=[
        pl.BlockSpec((R, D), lambda i: (i, 0)),
        pl.BlockSpec((R, D), lambda i: (i, 0)),
        pl.BlockSpec((1, D), lambda i: (0, 0)),
    ],
    out_specs=pl.BlockSpec((R, D), lambda i: (i, 0)),
    out_shape=jax.ShapeDtypeStruct((N, D), jnp.float32),
)


def kernel(x, edge_index, W1, b1, W2, b2):
    ei = edge_index.astype(jnp.int32)
    src, dst = ei[0], ei[1]
    src_a = src.reshape(NC, NS, A_C, A_K)
    dst_a = dst.reshape(NC, NS, A_C, A_K)
    src_g = src.reshape(NS, G_C, G_K)

    dinv = _deg_kernel(src_g)
    dcol = dinv[:N].reshape(N, 1)

    h1 = _lin1(x, W1, dcol)
    p = _agg_kernel(h1, src_a, dst_a)
    h2 = _lin2(p[0, :N], p[1, :N], b1.reshape(1, D), W2, dcol)
    q = _agg_kernel(h2, src_a, dst_a)
    return _fin(q[0, :N], q[1, :N], b2.reshape(1, D))
